# Initial kernel scaffold; baseline (speedup 1.0000x reference)
import jax
import jax.numpy as jnp
from jax.experimental import pallas as pl


def kernel(questions, e_s, answers, subj_idx, rel_idx, obj_idx, W_step0, b_step0, W_step1, b_step1, W_cq, b_cq, rel_emb, ent_emb, ent_bias):
    raise NotImplementedError("write your pallas kernel here")


# speedup vs baseline: 13.0231x; 13.0231x over previous
import functools

import jax
import jax.numpy as jnp
from jax import lax
from jax.experimental import pallas as pl
from jax.experimental.pallas import tpu as pltpu
from jax.experimental.pallas import tpu_sc as plsc

NE = 50000
EP = 50176
NR = 500
D = 128
B = 32
HB = 16
T = 800000
TP = 802816
CHUNKS = 392
PCH = 98
C = 128
RPT = EP // 16
EBLK = 512
EGRID = EP // EBLK


def _follow_body(e_hbm, r0_hbm, r1_hbm, sj_hbm, rl_hbm, ob_hbm, ans_hbm,
                 gt_hbm, out_hbm, e1_hbm,
                 sj_v, rl_v, ob_v, eg0, eg1, rg0, rg1, zb, wb, av, gv, acc,
                 semE0, semE1, semR0, semR1):
    cid = lax.axis_index("c")
    sid = lax.axis_index("s")

    pltpu.sync_copy(ans_hbm.at[cid], av)
    pltpu.sync_copy(gt_hbm.at[cid], gv)

    zeros16 = jnp.zeros((16,), jnp.float32)

    @pl.loop(0, 392)
    def _(i):
        zb[i, pl.ds(0, 16)] = zeros16

    @pl.loop(0, 8)
    def _(k):
        pltpu.sync_copy(zb, acc.at[pl.ds(sid * RPT + k * 392, 392)])

    plsc.subcore_barrier()

    def issue(e_src, r_src, j, eg, rg, semE, semR):
        pltpu.async_copy(e_src.at[sj_v.at[j]], eg, semE)
        pltpu.async_copy(r_src.at[rl_v.at[j]], rg, semR)

    def crunch(e_src, r_src, j, eg, rg, semE, semR):
        pltpu.make_async_copy(e_src.at[sj_v.at[j]], eg, semE).wait()
        pltpu.make_async_copy(r_src.at[rl_v.at[j]], rg, semR).wait()

        @plsc.parallel_loop(0, C, unroll=8)
        def _(i):
            eg[i, pl.ds(0, 16)] = eg[i, pl.ds(0, 16)] * rg[i, pl.ds(0, 16)]

        pltpu.sync_copy(eg, acc.at[ob_v.at[j]], add=True)

    def run_step(e_src, r_src):
        for phase in range(CHUNKS // PCH):
            pltpu.sync_copy(sj_hbm.at[sid, pl.ds(phase * PCH, PCH)], sj_v)
            pltpu.sync_copy(rl_hbm.at[sid, pl.ds(phase * PCH, PCH)], rl_v)
            pltpu.sync_copy(ob_hbm.at[sid, pl.ds(phase * PCH, PCH)], ob_v)

            issue(e_src, r_src, 0, eg0, rg0, semE0, semR0)

            @pl.loop(0, PCH // 2)
            def _(g):
                j0 = 2 * g
                j1 = 2 * g + 1

                issue(e_src, r_src, j1, eg1, rg1, semE1, semR1)
                crunch(e_src, r_src, j0, eg0, rg0, semE0, semR0)

                @pl.when(j1 + 1 < PCH)
                def _():
                    issue(e_src, r_src, j1 + 1, eg0, rg0, semE0, semR0)

                crunch(e_src, r_src, j1, eg1, rg1, semE1, semR1)

    run_step(e_hbm.at[cid], r0_hbm.at[cid])
    plsc.subcore_barrier()

    @pl.loop(0, 8)
    def _(k):
        base = sid * RPT + k * 392
        pltpu.sync_copy(acc.at[pl.ds(base, 392)], wb)

        @pl.loop(0, 392)
        def _(r):
            row = base + r
            v = wb[r, pl.ds(0, 16)]
            v = v - jnp.where(av[...] == row, gv[...], 0.0)
            wb[r, pl.ds(0, 16)] = jnp.minimum(v, 1.0)

        pltpu.sync_copy(wb, e1_hbm.at[cid, pl.ds(base, 392)])
        pltpu.sync_copy(zb, acc.at[pl.ds(base, 392)])

    plsc.subcore_barrier()

    run_step(e1_hbm.at[cid], r1_hbm.at[cid])
    plsc.subcore_barrier()

    pltpu.sync_copy(acc.at[pl.ds(sid * RPT, RPT)],
                    out_hbm.at[cid, pl.ds(sid * RPT, RPT)])


def _follow2(e2, r2_0, r2_1, sj3, rl3, ob3, ans_hb, gt_hb):
    f = functools.partial(
        pl.kernel,
        out_type=(jax.ShapeDtypeStruct((2, EP, HB), jnp.float32),
                  jax.ShapeDtypeStruct((2, EP, HB), jnp.float32)),
        mesh=plsc.VectorSubcoreMesh(core_axis_name="c", subcore_axis_name="s"),
        scratch_types=[
            pltpu.VMEM((PCH, C), jnp.int32),
            pltpu.VMEM((PCH, C), jnp.int32),
            pltpu.VMEM((PCH, C), jnp.int32),
            pltpu.VMEM((C, HB), jnp.float32),
            pltpu.VMEM((C, HB), jnp.float32),
            pltpu.VMEM((C, HB), jnp.float32),
            pltpu.VMEM((C, HB), jnp.float32),
            pltpu.VMEM((392, HB), jnp.float32),
            pltpu.VMEM((392, HB), jnp.float32),
            pltpu.VMEM((HB,), jnp.int32),
            pltpu.VMEM((HB,), jnp.float32),
            pltpu.VMEM_SHARED((EP, HB), jnp.float32),
            pltpu.SemaphoreType.DMA,
            pltpu.SemaphoreType.DMA,
            pltpu.SemaphoreType.DMA,
            pltpu.SemaphoreType.DMA,
        ],
        compiler_params=pltpu.CompilerParams(use_tc_tiling_on_sc=False),
    )(_follow_body)
    return f(e2, r2_0, r2_1, sj3, rl3, ob3, ans_hb, gt_hb)


def _passa_body(p_ref, emb_ref, ans_ref, head_ref, kill_ref,
                p1_ref, s_ref, ls_ref):
    j = pl.program_id(0)

    @pl.when(j == 0)
    def _():
        p1_ref[...] = jnp.zeros((B, D), jnp.float32)
        s_ref[...] = jnp.zeros((1, B), jnp.float32)
        ls_ref[...] = jnp.zeros((1, 1), jnp.float32)

    e = jnp.concatenate([p_ref[0], p_ref[1]], axis=1)
    e = jnp.minimum(e, 1.0)
    rows = lax.broadcasted_iota(jnp.int32, (EBLK, B), 0) + j * EBLK
    e = e * (1.0 - jnp.where(rows == head_ref[...], kill_ref[...], 0.0))
    e = jnp.where(rows < NE, e, 0.0)
    a = (rows == ans_ref[...]).astype(jnp.float32)
    w = a * 9.0 + 1.0
    ls_ref[...] += jnp.sum(w * (e - a) * (e - a))[None, None]
    s_ref[...] += jnp.sum(e, axis=0, keepdims=True)
    p1_ref[...] += lax.dot_general(e, emb_ref[...], (((0,), (0,)), ((), ())),
                                   preferred_element_type=jnp.float32)

    @pl.when(j == EGRID - 1)
    def _():
        ls_ref[...] = ls_ref[...] * (1.0 / (B * NE))


def _pass_a(parts, emb_p, ans2, head2, kill2):
    return pl.pallas_call(
        _passa_body,
        grid=(EGRID,),
        in_specs=[
            pl.BlockSpec((2, EBLK, HB), lambda j: (0, j, 0)),
            pl.BlockSpec((EBLK, D), lambda j: (j, 0)),
            pl.BlockSpec((1, B), lambda j: (0, 0)),
            pl.BlockSpec((1, B), lambda j: (0, 0)),
            pl.BlockSpec((1, B), lambda j: (0, 0)),
        ],
        out_specs=[
            pl.BlockSpec((B, D), lambda j: (0, 0)),
            pl.BlockSpec((1, B), lambda j: (0, 0)),
            pl.BlockSpec((1, 1), lambda j: (0, 0)),
        ],
        out_shape=[
            jax.ShapeDtypeStruct((B, D), jnp.float32),
            jax.ShapeDtypeStruct((1, B), jnp.float32),
            jax.ShapeDtypeStruct((1, 1), jnp.float32),
        ],
        compiler_params=pltpu.CompilerParams(
            dimension_semantics=("arbitrary",)),
    )(parts, emb_p, ans2, head2, kill2)


def _passb_body(p1_ref, invs_ref, emb_ref, bias_ref, ansc_ref,
                lp_ref, m_s, s_s, pa_s):
    j = pl.program_id(0)

    @pl.when(j == 0)
    def _():
        m_s[...] = jnp.full((B, 1), -1e30, jnp.float32)
        s_s[...] = jnp.zeros((B, 1), jnp.float32)
        pa_s[...] = jnp.zeros((B, 1), jnp.float32)

    p1n = p1_ref[...] * invs_ref[...]
    pred = lax.dot_general(p1n, emb_ref[...], (((1,), (1,)), ((), ())),
                           preferred_element_type=jnp.float32)
    pred = pred + bias_ref[...]
    cols = lax.broadcasted_iota(jnp.int32, (B, EBLK), 1) + j * EBLK
    pred = jnp.where(cols < NE, pred, -1e30)
    bm = jnp.max(pred, axis=1, keepdims=True)
    mnew = jnp.maximum(m_s[...], bm)
    s_s[...] = (s_s[...] * jnp.exp(m_s[...] - mnew)
                + jnp.sum(jnp.exp(pred - mnew), axis=1, keepdims=True))
    m_s[...] = mnew
    pa_s[...] += jnp.sum(jnp.where(cols == ansc_ref[...], pred, 0.0),
                         axis=1, keepdims=True)

    @pl.when(j == EGRID - 1)
    def _():
        lp_ref[...] = -jnp.mean(pa_s[...] - m_s[...] - jnp.log(s_s[...]))[None, None]


def _pass_b(p1_raw, inv_s, emb_p, bias2, ans_col):
    return pl.pallas_call(
        _passb_body,
        grid=(EGRID,),
        in_specs=[
            pl.BlockSpec((B, D), lambda j: (0, 0)),
            pl.BlockSpec((B, 1), lambda j: (0, 0)),
            pl.BlockSpec((EBLK, D), lambda j: (j, 0)),
            pl.BlockSpec((1, EBLK), lambda j: (0, j)),
            pl.BlockSpec((B, 1), lambda j: (0, 0)),
        ],
        out_specs=pl.BlockSpec((1, 1), lambda j: (0, 0)),
        out_shape=jax.ShapeDtypeStruct((1, 1), jnp.float32),
        scratch_shapes=[
            pltpu.VMEM((B, 1), jnp.float32),
            pltpu.VMEM((B, 1), jnp.float32),
            pltpu.VMEM((B, 1), jnp.float32),
        ],
        compiler_params=pltpu.CompilerParams(
            dimension_semantics=("arbitrary",)),
    )(p1_raw, inv_s, emb_p, bias2, ans_col)


def kernel(questions, e_s, answers, subj_idx, rel_idx, obj_idx,
           W_step0, b_step0, W_step1, b_step1, W_cq, b_cq,
           rel_emb, ent_emb, ent_bias):
    q = questions[:, 0]
    q_emb = rel_emb[q]
    rd0 = jax.nn.softmax(jnp.tanh(q_emb @ W_step0 + b_step0) @ rel_emb.T, axis=1)
    rd1 = jax.nn.softmax(jnp.tanh(q_emb @ W_step1 + b_step1) @ rel_emb.T, axis=1)
    gt_mask = jnp.take_along_axis(rd0, questions, axis=1)[:, 0]
    prev_rel = jnp.argmax(rd0, axis=1)
    curr_rel = jnp.argmax(rd1, axis=1)
    cond = ((jnp.abs(prev_rel - curr_rel) == 1)
            & (jnp.mod(jnp.minimum(prev_rel, curr_rel), 2) == 0))
    num_self = ((prev_rel == 0).astype(jnp.int32)
                + (curr_rel == 0).astype(jnp.int32))
    kill = jnp.maximum(cond.astype(jnp.float32),
                       (num_self == 1).astype(jnp.float32))
    head_ids = jnp.argmax(e_s, axis=1).astype(jnp.int32)
    ans_ids = jnp.argmax(answers, axis=1).astype(jnp.int32)

    r2_0 = jnp.stack([rd0[:HB].T, rd0[HB:].T])
    r2_1 = jnp.stack([rd1[:HB].T, rd1[HB:].T])
    zpadr = ((0, EP - NE), (0, 0))
    e2_0 = jnp.stack([jnp.pad(e_s[:HB].T, zpadr),
                      jnp.pad(e_s[HB:].T, zpadr)])
    emb_p = jnp.pad(ent_emb, zpadr)
    bias2 = jnp.pad(ent_bias, (0, EP - NE)).reshape(1, EP)

    pad = TP - T
    zpad = jnp.zeros((pad,), jnp.int32)
    sj3 = jnp.concatenate([subj_idx, zpad]).reshape(16, CHUNKS, C)
    rl3 = jnp.concatenate([rel_idx, zpad]).reshape(16, CHUNKS, C)
    opad = NE + (jnp.arange(pad, dtype=jnp.int32) % (EP - NE))
    ob3 = jnp.concatenate([obj_idx, opad]).reshape(16, CHUNKS, C)

    ans2 = ans_ids.reshape(1, B)
    head2 = head_ids.reshape(1, B)
    kill2 = kill.reshape(1, B)

    parts1, _e1 = _follow2(e2_0, r2_0, r2_1, sj3, rl3, ob3,
                           ans_ids.reshape(2, HB), gt_mask.reshape(2, HB))

    p1_raw, s_sum, ls = _pass_a(parts1, emb_p, ans2, head2, kill2)
    inv_s = (1.0 / (s_sum + 1e-6)).reshape(B, 1)
    lp = _pass_b(p1_raw, inv_s, emb_p, bias2, ans_ids.reshape(B, 1))

    return (ls[0, 0], lp[0, 0])


# speedup vs baseline: 16.5569x; 1.2714x over previous
import functools

import jax
import jax.numpy as jnp
from jax import lax
from jax.experimental import pallas as pl
from jax.experimental.pallas import tpu as pltpu
from jax.experimental.pallas import tpu_sc as plsc

NE = 50000
EP = 50176
NR = 500
D = 128
B = 32
HB = 16
T = 800000
TP = 802816
CHUNKS = 392
PCH = 56
NBUF = 4
C = 128
RPT = EP // 16
EBLK = 512
EGRID = EP // EBLK


def _follow_body(r0_hbm, r1_hbm, sj_hbm, rl_hbm, ob_hbm, head_hbm, ans_hbm,
                 gt_hbm, out_hbm, e1_hbm,
                 sj_v, rl_v, ob_v, eg0, eg1, eg2, eg3, rg0, rg1, rg2, rg3,
                 pr0, pr1, zb, wb, hv, av, gv, acc,
                 semE0, semE1, semE2, semE3, semR0, semR1, semR2, semR3,
                 semS0, semS1):
    egs = (eg0, eg1, eg2, eg3)
    rgs = (rg0, rg1, rg2, rg3)
    semsE = (semE0, semE1, semE2, semE3)
    semsR = (semR0, semR1, semR2, semR3)
    cid = lax.axis_index("c")
    sid = lax.axis_index("s")

    pltpu.sync_copy(head_hbm.at[cid], hv)
    pltpu.sync_copy(ans_hbm.at[cid], av)
    pltpu.sync_copy(gt_hbm.at[cid], gv)

    zeros16 = jnp.zeros((16,), jnp.float32)

    @pl.loop(0, 392)
    def _(i):
        zb[i, pl.ds(0, 16)] = zeros16

    @pl.loop(0, 8)
    def _(k):
        pltpu.sync_copy(zb, acc.at[pl.ds(sid * RPT + k * 392, 392)])

    plsc.subcore_barrier()

    prods = (pr0, pr1)
    sems_s = (semS0, semS1)

    def wait_scatter(p, j):
        pltpu.make_async_copy(prods[p], acc.at[ob_v.at[j]], sems_s[p]).wait()

    def stage_idx(phase):
        pltpu.sync_copy(sj_hbm.at[sid, pl.ds(phase * PCH, PCH)], sj_v)
        pltpu.sync_copy(rl_hbm.at[sid, pl.ds(phase * PCH, PCH)], rl_v)
        pltpu.sync_copy(ob_hbm.at[sid, pl.ds(phase * PCH, PCH)], ob_v)

    def run_step(e_src, r_src, crunch):
        def issue(j, b):
            if e_src is not None:
                pltpu.async_copy(e_src.at[sj_v.at[j]], egs[b], semsE[b])
            pltpu.async_copy(r_src.at[rl_v.at[j]], rgs[b], semsR[b])

        def wait_gather(j, b):
            if e_src is not None:
                pltpu.make_async_copy(e_src.at[sj_v.at[j]], egs[b],
                                      semsE[b]).wait()
            pltpu.make_async_copy(r_src.at[rl_v.at[j]], rgs[b],
                                  semsR[b]).wait()

        for phase in range(CHUNKS // PCH):
            stage_idx(phase)
            for b in range(NBUF - 1):
                issue(b, b)

            @pl.loop(0, PCH // NBUF)
            def _(g):
                for b in range(NBUF):
                    j = NBUF * g + b
                    p = b % 2

                    wait_gather(j, b)

                    @pl.when(j >= 2)
                    def _():
                        wait_scatter(p, j - 2)

                    crunch(j, b, p)

                    @pl.when(j + NBUF - 1 < PCH)
                    def _():
                        issue(j + NBUF - 1, (b + NBUF - 1) % NBUF)

                    pltpu.async_copy(prods[p], acc.at[ob_v.at[j]], sems_s[p],
                                     add=True)

            wait_scatter(0, PCH - 2)
            wait_scatter(1, PCH - 1)

    def crunch0(j, b, p):
        rg = rgs[b]
        pr = prods[p]
        hvv = hv[...]

        @plsc.parallel_loop(0, C, unroll=8)
        def _(i):
            sv = plsc.load_gather(
                sj_v, [jnp.full((16,), j, jnp.int32),
                       jnp.full((16,), i, jnp.int32)])
            pr[i, pl.ds(0, 16)] = jnp.where(sv == hvv, rg[i, pl.ds(0, 16)],
                                            0.0)

    run_step(None, r0_hbm.at[cid], crunch0)
    plsc.subcore_barrier()

    @pl.loop(0, 8)
    def _(k):
        base = sid * RPT + k * 392
        pltpu.sync_copy(acc.at[pl.ds(base, 392)], wb)

        @pl.loop(0, 392)
        def _(r):
            row = base + r
            v = wb[r, pl.ds(0, 16)]
            v = v - jnp.where(av[...] == row, gv[...], 0.0)
            wb[r, pl.ds(0, 16)] = jnp.minimum(v, 1.0)

        pltpu.sync_copy(wb, e1_hbm.at[cid, pl.ds(base, 392)])
        pltpu.sync_copy(zb, acc.at[pl.ds(base, 392)])

    plsc.subcore_barrier()

    def crunch1(j, b, p):
        eg = egs[b]
        rg = rgs[b]
        pr = prods[p]

        @plsc.parallel_loop(0, C, unroll=8)
        def _(i):
            pr[i, pl.ds(0, 16)] = (eg[i, pl.ds(0, 16)]
                                   * rg[i, pl.ds(0, 16)])

    run_step(e1_hbm.at[cid], r1_hbm.at[cid], crunch1)
    plsc.subcore_barrier()

    pltpu.sync_copy(acc.at[pl.ds(sid * RPT, RPT)],
                    out_hbm.at[cid, pl.ds(sid * RPT, RPT)])


def _follow2(r2_0, r2_1, sj3, rl3, ob3, head_hb, ans_hb, gt_hb):
    f = functools.partial(
        pl.kernel,
        out_type=(jax.ShapeDtypeStruct((2, EP, HB), jnp.float32),
                  jax.ShapeDtypeStruct((2, EP, HB), jnp.float32)),
        mesh=plsc.VectorSubcoreMesh(core_axis_name="c", subcore_axis_name="s"),
        scratch_types=(
            [pltpu.VMEM((PCH, C), jnp.int32)] * 3
            + [pltpu.VMEM((C, HB), jnp.float32)] * (2 * NBUF + 2)
            + [pltpu.VMEM((392, HB), jnp.float32)] * 2
            + [pltpu.VMEM((HB,), jnp.int32),
               pltpu.VMEM((HB,), jnp.int32),
               pltpu.VMEM((HB,), jnp.float32),
               pltpu.VMEM_SHARED((EP, HB), jnp.float32)]
            + [pltpu.SemaphoreType.DMA] * (2 * NBUF + 2)
        ),
        compiler_params=pltpu.CompilerParams(use_tc_tiling_on_sc=False,
                                             needs_layout_passes=False),
    )(_follow_body)
    return f(r2_0, r2_1, sj3, rl3, ob3, head_hb, ans_hb, gt_hb)


def _passa_body(p_ref, emb_ref, ans_ref, head_ref, kill_ref,
                p1_ref, s_ref, ls_ref):
    j = pl.program_id(0)

    @pl.when(j == 0)
    def _():
        p1_ref[...] = jnp.zeros((B, D), jnp.float32)
        s_ref[...] = jnp.zeros((1, B), jnp.float32)
        ls_ref[...] = jnp.zeros((1, 1), jnp.float32)

    e = jnp.concatenate([p_ref[0], p_ref[1]], axis=1)
    e = jnp.minimum(e, 1.0)
    rows = lax.broadcasted_iota(jnp.int32, (EBLK, B), 0) + j * EBLK
    e = e * (1.0 - jnp.where(rows == head_ref[...], kill_ref[...], 0.0))
    e = jnp.where(rows < NE, e, 0.0)
    a = (rows == ans_ref[...]).astype(jnp.float32)
    w = a * 9.0 + 1.0
    ls_ref[...] += jnp.sum(w * (e - a) * (e - a))[None, None]
    s_ref[...] += jnp.sum(e, axis=0, keepdims=True)
    p1_ref[...] += lax.dot_general(e, emb_ref[...], (((0,), (0,)), ((), ())),
                                   preferred_element_type=jnp.float32)

    @pl.when(j == EGRID - 1)
    def _():
        ls_ref[...] = ls_ref[...] * (1.0 / (B * NE))


def _pass_a(parts, emb_p, ans2, head2, kill2):
    return pl.pallas_call(
        _passa_body,
        grid=(EGRID,),
        in_specs=[
            pl.BlockSpec((2, EBLK, HB), lambda j: (0, j, 0)),
            pl.BlockSpec((EBLK, D), lambda j: (j, 0)),
            pl.BlockSpec((1, B), lambda j: (0, 0)),
            pl.BlockSpec((1, B), lambda j: (0, 0)),
            pl.BlockSpec((1, B), lambda j: (0, 0)),
        ],
        out_specs=[
            pl.BlockSpec((B, D), lambda j: (0, 0)),
            pl.BlockSpec((1, B), lambda j: (0, 0)),
            pl.BlockSpec((1, 1), lambda j: (0, 0)),
        ],
        out_shape=[
            jax.ShapeDtypeStruct((B, D), jnp.float32),
            jax.ShapeDtypeStruct((1, B), jnp.float32),
            jax.ShapeDtypeStruct((1, 1), jnp.float32),
        ],
        compiler_params=pltpu.CompilerParams(
            dimension_semantics=("arbitrary",)),
    )(parts, emb_p, ans2, head2, kill2)


def _passb_body(p1_ref, invs_ref, emb_ref, bias_ref, ansc_ref,
                lp_ref, m_s, s_s, pa_s):
    j = pl.program_id(0)

    @pl.when(j == 0)
    def _():
        m_s[...] = jnp.full((B, 1), -1e30, jnp.float32)
        s_s[...] = jnp.zeros((B, 1), jnp.float32)
        pa_s[...] = jnp.zeros((B, 1), jnp.float32)

    p1n = p1_ref[...] * invs_ref[...]
    pred = lax.dot_general(p1n, emb_ref[...], (((1,), (1,)), ((), ())),
                           preferred_element_type=jnp.float32)
    pred = pred + bias_ref[...]
    cols = lax.broadcasted_iota(jnp.int32, (B, EBLK), 1) + j * EBLK
    pred = jnp.where(cols < NE, pred, -1e30)
    bm = jnp.max(pred, axis=1, keepdims=True)
    mnew = jnp.maximum(m_s[...], bm)
    s_s[...] = (s_s[...] * jnp.exp(m_s[...] - mnew)
                + jnp.sum(jnp.exp(pred - mnew), axis=1, keepdims=True))
    m_s[...] = mnew
    pa_s[...] += jnp.sum(jnp.where(cols == ansc_ref[...], pred, 0.0),
                         axis=1, keepdims=True)

    @pl.when(j == EGRID - 1)
    def _():
        lp_ref[...] = -jnp.mean(pa_s[...] - m_s[...] - jnp.log(s_s[...]))[None, None]


def _pass_b(p1_raw, inv_s, emb_p, bias2, ans_col):
    return pl.pallas_call(
        _passb_body,
        grid=(EGRID,),
        in_specs=[
            pl.BlockSpec((B, D), lambda j: (0, 0)),
            pl.BlockSpec((B, 1), lambda j: (0, 0)),
            pl.BlockSpec((EBLK, D), lambda j: (j, 0)),
            pl.BlockSpec((1, EBLK), lambda j: (0, j)),
            pl.BlockSpec((B, 1), lambda j: (0, 0)),
        ],
        out_specs=pl.BlockSpec((1, 1), lambda j: (0, 0)),
        out_shape=jax.ShapeDtypeStruct((1, 1), jnp.float32),
        scratch_shapes=[
            pltpu.VMEM((B, 1), jnp.float32),
            pltpu.VMEM((B, 1), jnp.float32),
            pltpu.VMEM((B, 1), jnp.float32),
        ],
        compiler_params=pltpu.CompilerParams(
            dimension_semantics=("arbitrary",)),
    )(p1_raw, inv_s, emb_p, bias2, ans_col)


def kernel(questions, e_s, answers, subj_idx, rel_idx, obj_idx,
           W_step0, b_step0, W_step1, b_step1, W_cq, b_cq,
           rel_emb, ent_emb, ent_bias):
    q = questions[:, 0]
    q_emb = rel_emb[q]
    rd0 = jax.nn.softmax(jnp.tanh(q_emb @ W_step0 + b_step0) @ rel_emb.T, axis=1)
    rd1 = jax.nn.softmax(jnp.tanh(q_emb @ W_step1 + b_step1) @ rel_emb.T, axis=1)
    gt_mask = jnp.take_along_axis(rd0, questions, axis=1)[:, 0]
    prev_rel = jnp.argmax(rd0, axis=1)
    curr_rel = jnp.argmax(rd1, axis=1)
    cond = ((jnp.abs(prev_rel - curr_rel) == 1)
            & (jnp.mod(jnp.minimum(prev_rel, curr_rel), 2) == 0))
    num_self = ((prev_rel == 0).astype(jnp.int32)
                + (curr_rel == 0).astype(jnp.int32))
    kill = jnp.maximum(cond.astype(jnp.float32),
                       (num_self == 1).astype(jnp.float32))
    head_ids = jnp.argmax(e_s, axis=1).astype(jnp.int32)
    ans_ids = jnp.argmax(answers, axis=1).astype(jnp.int32)

    r2_0 = jnp.stack([rd0[:HB].T, rd0[HB:].T])
    r2_1 = jnp.stack([rd1[:HB].T, rd1[HB:].T])
    zpadr = ((0, EP - NE), (0, 0))
    emb_p = jnp.pad(ent_emb, zpadr)
    bias2 = jnp.pad(ent_bias, (0, EP - NE)).reshape(1, EP)

    pad = TP - T
    zpad = jnp.zeros((pad,), jnp.int32)
    sj3 = jnp.concatenate([subj_idx, zpad]).reshape(16, CHUNKS, C)
    rl3 = jnp.concatenate([rel_idx, zpad]).reshape(16, CHUNKS, C)
    opad = NE + (jnp.arange(pad, dtype=jnp.int32) % (EP - NE))
    ob3 = jnp.concatenate([obj_idx, opad]).reshape(16, CHUNKS, C)

    ans2 = ans_ids.reshape(1, B)
    head2 = head_ids.reshape(1, B)
    kill2 = kill.reshape(1, B)

    parts1, _e1 = _follow2(r2_0, r2_1, sj3, rl3, ob3,
                           head_ids.reshape(2, HB), ans_ids.reshape(2, HB),
                           gt_mask.reshape(2, HB))

    p1_raw, s_sum, ls = _pass_a(parts1, emb_p, ans2, head2, kill2)
    inv_s = (1.0 / (s_sum + 1e-6)).reshape(B, 1)
    lp = _pass_b(p1_raw, inv_s, emb_p, bias2, ans_ids.reshape(B, 1))

    return (ls[0, 0], lp[0, 0])


# speedup vs baseline: 18.8968x; 1.1413x over previous
import functools

import jax
import jax.numpy as jnp
from jax import lax
from jax.experimental import pallas as pl
from jax.experimental.pallas import tpu as pltpu
from jax.experimental.pallas import tpu_sc as plsc

NE = 50000
EP = 50176
NR = 500
D = 128
B = 32
HB = 16
T = 800000
TP = 802816
CHUNKS = 392
PCH = 56
NBUF = 8
BMW = EP // 32
C = 128
RPT = EP // 16
EBLK = 512
EGRID = EP // EBLK


def _follow_body(r0_hbm, r1_hbm, sj_hbm, rl_hbm, ob_hbm, head_hbm, ans_hbm,
                 gt_hbm, bm_hbm, out_hbm, e1_hbm,
                 sj_v, rl_v, ob_v,
                 eg0, eg1, eg2, eg3, eg4, eg5, eg6, eg7,
                 rg0, rg1, rg2, rg3, rg4, rg5, rg6, rg7,
                 pr0, pr1, zb, wb, hv, av, gv, bmv, acc,
                 semE0, semE1, semE2, semE3, semE4, semE5, semE6, semE7,
                 semR0, semR1, semR2, semR3, semR4, semR5, semR6, semR7,
                 semS0, semS1):
    egs = (eg0, eg1, eg2, eg3, eg4, eg5, eg6, eg7)
    rgs = (rg0, rg1, rg2, rg3, rg4, rg5, rg6, rg7)
    semsE = (semE0, semE1, semE2, semE3, semE4, semE5, semE6, semE7)
    semsR = (semR0, semR1, semR2, semR3, semR4, semR5, semR6, semR7)
    cid = lax.axis_index("c")
    sid = lax.axis_index("s")

    pltpu.sync_copy(head_hbm.at[cid], hv)
    pltpu.sync_copy(ans_hbm.at[cid], av)
    pltpu.sync_copy(gt_hbm.at[cid], gv)
    pltpu.sync_copy(bm_hbm.at[cid], bmv)

    zeros16 = jnp.zeros((16,), jnp.float32)

    @pl.loop(0, 392)
    def _(i):
        zb[i, pl.ds(0, 16)] = zeros16

    @pl.loop(0, 8)
    def _(k):
        pltpu.sync_copy(zb, acc.at[pl.ds(sid * RPT + k * 392, 392)])

    plsc.subcore_barrier()

    prods = (pr0, pr1)
    sems_s = (semS0, semS1)

    def wait_scatter(p, j):
        pltpu.make_async_copy(prods[p], acc.at[ob_v.at[j]], sems_s[p]).wait()

    def stage_idx(phase):
        pltpu.sync_copy(sj_hbm.at[sid, pl.ds(phase * PCH, PCH)], sj_v)
        pltpu.sync_copy(rl_hbm.at[sid, pl.ds(phase * PCH, PCH)], rl_v)
        pltpu.sync_copy(ob_hbm.at[sid, pl.ds(phase * PCH, PCH)], ob_v)

    def run_step(e_src, r_src, crunch):
        def issue(j, b):
            if e_src is not None:
                pltpu.async_copy(e_src.at[sj_v.at[j]], egs[b], semsE[b])
            pltpu.async_copy(r_src.at[rl_v.at[j]], rgs[b], semsR[b])

        def wait_gather(j, b):
            if e_src is not None:
                pltpu.make_async_copy(e_src.at[sj_v.at[j]], egs[b],
                                      semsE[b]).wait()
            pltpu.make_async_copy(r_src.at[rl_v.at[j]], rgs[b],
                                  semsR[b]).wait()

        for phase in range(CHUNKS // PCH):
            stage_idx(phase)
            for b in range(NBUF - 1):
                issue(b, b)

            @pl.loop(0, PCH // NBUF)
            def _(g):
                for b in range(NBUF):
                    j = NBUF * g + b
                    p = b % 2

                    wait_gather(j, b)

                    @pl.when(j >= 2)
                    def _():
                        wait_scatter(p, j - 2)

                    crunch(j, b, p)

                    @pl.when(j + NBUF - 1 < PCH)
                    def _():
                        issue(j + NBUF - 1, (b + NBUF - 1) % NBUF)

                    pltpu.async_copy(prods[p], acc.at[ob_v.at[j]], sems_s[p],
                                     add=True)

            wait_scatter(0, PCH - 2)
            wait_scatter(1, PCH - 1)

    hvv = hv[...]

    for phase in range(CHUNKS // PCH):
        stage_idx(phase)

        @pl.loop(0, PCH)
        def _(j):
            def grp(k, m):
                sv = sj_v[j, pl.ds(k * 16, 16)]
                w = plsc.load_gather(bmv, [lax.shift_right_logical(sv, 5)])
                return m | lax.shift_right_logical(w, sv & 31)

            m = lax.fori_loop(0, C // 16, grp, jnp.zeros((16,), jnp.int32))
            flag = jnp.max(m & 1)

            @pl.when(flag != 0)
            def _():
                pltpu.sync_copy(r0_hbm.at[cid].at[rl_v.at[j]], rg0)

                @plsc.parallel_loop(0, C, unroll=8)
                def _(i):
                    sv = plsc.load_gather(
                        sj_v, [jnp.full((16,), j, jnp.int32),
                               jnp.full((16,), i, jnp.int32)])
                    pr0[i, pl.ds(0, 16)] = jnp.where(
                        sv == hvv, rg0[i, pl.ds(0, 16)], 0.0)

                pltpu.sync_copy(pr0, acc.at[ob_v.at[j]], add=True)

    plsc.subcore_barrier()

    @pl.loop(0, 8)
    def _(k):
        base = sid * RPT + k * 392
        pltpu.sync_copy(acc.at[pl.ds(base, 392)], wb)

        @pl.loop(0, 392)
        def _(r):
            row = base + r
            v = wb[r, pl.ds(0, 16)]
            v = v - jnp.where(av[...] == row, gv[...], 0.0)
            wb[r, pl.ds(0, 16)] = jnp.minimum(v, 1.0)

        pltpu.sync_copy(wb, e1_hbm.at[cid, pl.ds(base, 392)])
        pltpu.sync_copy(zb, acc.at[pl.ds(base, 392)])

    plsc.subcore_barrier()

    def crunch1(j, b, p):
        eg = egs[b]
        rg = rgs[b]
        pr = prods[p]

        @plsc.parallel_loop(0, C, unroll=8)
        def _(i):
            pr[i, pl.ds(0, 16)] = (eg[i, pl.ds(0, 16)]
                                   * rg[i, pl.ds(0, 16)])

    run_step(e1_hbm.at[cid], r1_hbm.at[cid], crunch1)
    plsc.subcore_barrier()

    pltpu.sync_copy(acc.at[pl.ds(sid * RPT, RPT)],
                    out_hbm.at[cid, pl.ds(sid * RPT, RPT)])


def _follow2(r2_0, r2_1, sj3, rl3, ob3, head_hb, ans_hb, gt_hb, bm):
    f = functools.partial(
        pl.kernel,
        out_type=(jax.ShapeDtypeStruct((2, EP, HB), jnp.float32),
                  jax.ShapeDtypeStruct((2, EP, HB), jnp.float32)),
        mesh=plsc.VectorSubcoreMesh(core_axis_name="c", subcore_axis_name="s"),
        scratch_types=(
            [pltpu.VMEM((PCH, C), jnp.int32)] * 3
            + [pltpu.VMEM((C, HB), jnp.float32)] * (2 * NBUF + 2)
            + [pltpu.VMEM((392, HB), jnp.float32)] * 2
            + [pltpu.VMEM((HB,), jnp.int32),
               pltpu.VMEM((HB,), jnp.int32),
               pltpu.VMEM((HB,), jnp.float32),
               pltpu.VMEM((BMW,), jnp.int32),
               pltpu.VMEM_SHARED((EP, HB), jnp.float32)]
            + [pltpu.SemaphoreType.DMA] * (2 * NBUF + 2)
        ),
        compiler_params=pltpu.CompilerParams(use_tc_tiling_on_sc=False,
                                             needs_layout_passes=False),
    )(_follow_body)
    return f(r2_0, r2_1, sj3, rl3, ob3, head_hb, ans_hb, gt_hb, bm)


def _passa_body(p_ref, emb_ref, ans_ref, head_ref, kill_ref,
                p1_ref, s_ref, ls_ref):
    j = pl.program_id(0)

    @pl.when(j == 0)
    def _():
        p1_ref[...] = jnp.zeros((B, D), jnp.float32)
        s_ref[...] = jnp.zeros((1, B), jnp.float32)
        ls_ref[...] = jnp.zeros((1, 1), jnp.float32)

    e = jnp.concatenate([p_ref[0], p_ref[1]], axis=1)
    e = jnp.minimum(e, 1.0)
    rows = lax.broadcasted_iota(jnp.int32, (EBLK, B), 0) + j * EBLK
    e = e * (1.0 - jnp.where(rows == head_ref[...], kill_ref[...], 0.0))
    e = jnp.where(rows < NE, e, 0.0)
    a = (rows == ans_ref[...]).astype(jnp.float32)
    w = a * 9.0 + 1.0
    ls_ref[...] += jnp.sum(w * (e - a) * (e - a))[None, None]
    s_ref[...] += jnp.sum(e, axis=0, keepdims=True)
    rows2 = lax.broadcasted_iota(jnp.int32, (EBLK, D), 0) + j * EBLK
    emb = jnp.where(rows2 < NE, emb_ref[...], 0.0)
    p1_ref[...] += lax.dot_general(e, emb, (((0,), (0,)), ((), ())),
                                   preferred_element_type=jnp.float32)

    @pl.when(j == EGRID - 1)
    def _():
        ls_ref[...] = ls_ref[...] * (1.0 / (B * NE))


def _pass_a(parts, emb_p, ans2, head2, kill2):
    return pl.pallas_call(
        _passa_body,
        grid=(EGRID,),
        in_specs=[
            pl.BlockSpec((2, EBLK, HB), lambda j: (0, j, 0)),
            pl.BlockSpec((EBLK, D), lambda j: (j, 0)),
            pl.BlockSpec((1, B), lambda j: (0, 0)),
            pl.BlockSpec((1, B), lambda j: (0, 0)),
            pl.BlockSpec((1, B), lambda j: (0, 0)),
        ],
        out_specs=[
            pl.BlockSpec((B, D), lambda j: (0, 0)),
            pl.BlockSpec((1, B), lambda j: (0, 0)),
            pl.BlockSpec((1, 1), lambda j: (0, 0)),
        ],
        out_shape=[
            jax.ShapeDtypeStruct((B, D), jnp.float32),
            jax.ShapeDtypeStruct((1, B), jnp.float32),
            jax.ShapeDtypeStruct((1, 1), jnp.float32),
        ],
        compiler_params=pltpu.CompilerParams(
            dimension_semantics=("arbitrary",)),
    )(parts, emb_p, ans2, head2, kill2)


def _passb_body(p1_ref, invs_ref, emb_ref, bias_ref, ansc_ref,
                lp_ref, m_s, s_s, pa_s):
    j = pl.program_id(0)

    @pl.when(j == 0)
    def _():
        m_s[...] = jnp.full((B, 1), -1e30, jnp.float32)
        s_s[...] = jnp.zeros((B, 1), jnp.float32)
        pa_s[...] = jnp.zeros((B, 1), jnp.float32)

    p1n = p1_ref[...] * invs_ref[...]
    pred = lax.dot_general(p1n, emb_ref[...], (((1,), (1,)), ((), ())),
                           preferred_element_type=jnp.float32)
    pred = pred + bias_ref[...]
    cols = lax.broadcasted_iota(jnp.int32, (B, EBLK), 1) + j * EBLK
    pred = jnp.where(cols < NE, pred, -1e30)
    bm = jnp.max(pred, axis=1, keepdims=True)
    mnew = jnp.maximum(m_s[...], bm)
    s_s[...] = (s_s[...] * jnp.exp(m_s[...] - mnew)
                + jnp.sum(jnp.exp(pred - mnew), axis=1, keepdims=True))
    m_s[...] = mnew
    pa_s[...] += jnp.sum(jnp.where(cols == ansc_ref[...], pred, 0.0),
                         axis=1, keepdims=True)

    @pl.when(j == EGRID - 1)
    def _():
        lp_ref[...] = -jnp.mean(pa_s[...] - m_s[...] - jnp.log(s_s[...]))[None, None]


def _pass_b(p1_raw, inv_s, emb_p, bias2, ans_col):
    return pl.pallas_call(
        _passb_body,
        grid=(EGRID,),
        in_specs=[
            pl.BlockSpec((B, D), lambda j: (0, 0)),
            pl.BlockSpec((B, 1), lambda j: (0, 0)),
            pl.BlockSpec((EBLK, D), lambda j: (j, 0)),
            pl.BlockSpec((1, EBLK), lambda j: (0, j)),
            pl.BlockSpec((B, 1), lambda j: (0, 0)),
        ],
        out_specs=pl.BlockSpec((1, 1), lambda j: (0, 0)),
        out_shape=jax.ShapeDtypeStruct((1, 1), jnp.float32),
        scratch_shapes=[
            pltpu.VMEM((B, 1), jnp.float32),
            pltpu.VMEM((B, 1), jnp.float32),
            pltpu.VMEM((B, 1), jnp.float32),
        ],
        compiler_params=pltpu.CompilerParams(
            dimension_semantics=("arbitrary",)),
    )(p1_raw, inv_s, emb_p, bias2, ans_col)


def kernel(questions, e_s, answers, subj_idx, rel_idx, obj_idx,
           W_step0, b_step0, W_step1, b_step1, W_cq, b_cq,
           rel_emb, ent_emb, ent_bias):
    q = questions[:, 0]
    q_emb = rel_emb[q]
    rd0 = jax.nn.softmax(jnp.tanh(q_emb @ W_step0 + b_step0) @ rel_emb.T, axis=1)
    rd1 = jax.nn.softmax(jnp.tanh(q_emb @ W_step1 + b_step1) @ rel_emb.T, axis=1)
    gt_mask = jnp.take_along_axis(rd0, questions, axis=1)[:, 0]
    prev_rel = jnp.argmax(rd0, axis=1)
    curr_rel = jnp.argmax(rd1, axis=1)
    cond = ((jnp.abs(prev_rel - curr_rel) == 1)
            & (jnp.mod(jnp.minimum(prev_rel, curr_rel), 2) == 0))
    num_self = ((prev_rel == 0).astype(jnp.int32)
                + (curr_rel == 0).astype(jnp.int32))
    kill = jnp.maximum(cond.astype(jnp.float32),
                       (num_self == 1).astype(jnp.float32))
    head_ids = jnp.argmax(e_s, axis=1).astype(jnp.int32)
    ans_ids = jnp.argmax(answers, axis=1).astype(jnp.int32)

    r2_0 = jnp.stack([rd0[:HB].T, rd0[HB:].T])
    r2_1 = jnp.stack([rd1[:HB].T, rd1[HB:].T])
    bias2 = ent_bias.reshape(1, NE)

    def mk_bitmap(h):
        hs = jnp.sort(h)
        uniq = jnp.concatenate([jnp.ones((1,), bool), hs[1:] != hs[:-1]])
        vals = jnp.where(uniq, jnp.left_shift(jnp.int32(1), hs % 32), 0)
        return jax.ops.segment_sum(vals, hs // 32, num_segments=BMW)

    bm = jnp.stack([mk_bitmap(head_ids[:HB]), mk_bitmap(head_ids[HB:])])

    pad = TP - T
    zpad = jnp.zeros((pad,), jnp.int32)
    sj3 = jnp.concatenate([subj_idx, zpad]).reshape(16, CHUNKS, C)
    rl3 = jnp.concatenate([rel_idx, zpad]).reshape(16, CHUNKS, C)
    opad = NE + (jnp.arange(pad, dtype=jnp.int32) % (EP - NE))
    ob3 = jnp.concatenate([obj_idx, opad]).reshape(16, CHUNKS, C)

    ans2 = ans_ids.reshape(1, B)
    head2 = head_ids.reshape(1, B)
    kill2 = kill.reshape(1, B)

    parts1, _e1 = _follow2(r2_0, r2_1, sj3, rl3, ob3,
                           head_ids.reshape(2, HB), ans_ids.reshape(2, HB),
                           gt_mask.reshape(2, HB), bm)

    p1_raw, s_sum, ls = _pass_a(parts1, ent_emb, ans2, head2, kill2)
    inv_s = (1.0 / (s_sum + 1e-6)).reshape(B, 1)
    lp = _pass_b(p1_raw, inv_s, ent_emb, bias2, ans_ids.reshape(B, 1))

    return (ls[0, 0], lp[0, 0])


# speedup vs baseline: 23.9863x; 1.2693x over previous
import functools

import jax
import jax.numpy as jnp
from jax import lax
from jax.experimental import pallas as pl
from jax.experimental.pallas import tpu as pltpu
from jax.experimental.pallas import tpu_sc as plsc

NE = 50000
EP = 50176
NR = 500
D = 128
B = 32
HB = 16
T = 800000
TP = 802816
CHUNKS = 392
PCH = 56
NBUF = 8
BMW = EP // 32
C = 128
RPT = EP // 16
EBLK = 1792
EGRID = EP // EBLK


def _follow_body(r0_hbm, r1_hbm, sj_hbm, rl_hbm, ob_hbm, head_hbm, ans_hbm,
                 gt_hbm, bm_hbm, out_hbm, e1_hbm,
                 sj_v, rl_v, ob_v,
                 eg0, eg1, eg2, eg3, eg4, eg5, eg6, eg7,
                 rg0, rg1, rg2, rg3, rg4, rg5, rg6, rg7,
                 pr0, pr1, zb, wb, hv, av, gv, bmv, acc,
                 semE0, semE1, semE2, semE3, semE4, semE5, semE6, semE7,
                 semR0, semR1, semR2, semR3, semR4, semR5, semR6, semR7,
                 semS0, semS1):
    egs = (eg0, eg1, eg2, eg3, eg4, eg5, eg6, eg7)
    rgs = (rg0, rg1, rg2, rg3, rg4, rg5, rg6, rg7)
    semsE = (semE0, semE1, semE2, semE3, semE4, semE5, semE6, semE7)
    semsR = (semR0, semR1, semR2, semR3, semR4, semR5, semR6, semR7)
    cid = lax.axis_index("c")
    sid = lax.axis_index("s")

    pltpu.sync_copy(head_hbm.at[cid], hv)
    pltpu.sync_copy(ans_hbm.at[cid], av)
    pltpu.sync_copy(gt_hbm.at[cid], gv)
    pltpu.sync_copy(bm_hbm.at[cid], bmv)

    zeros16 = jnp.zeros((16,), jnp.float32)

    @pl.loop(0, 392)
    def _(i):
        zb[i, pl.ds(0, 16)] = zeros16

    @pl.loop(0, 8)
    def _(k):
        pltpu.sync_copy(zb, acc.at[pl.ds(sid * RPT + k * 392, 392)])

    plsc.subcore_barrier()

    prods = (pr0, pr1)
    sems_s = (semS0, semS1)

    def wait_scatter(p, j):
        pltpu.make_async_copy(prods[p], acc.at[ob_v.at[j]], sems_s[p]).wait()

    def stage_idx(phase):
        pltpu.sync_copy(sj_hbm.at[sid, pl.ds(phase * PCH, PCH)], sj_v)
        pltpu.sync_copy(rl_hbm.at[sid, pl.ds(phase * PCH, PCH)], rl_v)
        pltpu.sync_copy(ob_hbm.at[sid, pl.ds(phase * PCH, PCH)], ob_v)

    def run_step(e_src, r_src, crunch):
        def issue(j, b):
            if e_src is not None:
                pltpu.async_copy(e_src.at[sj_v.at[j]], egs[b], semsE[b])
            pltpu.async_copy(r_src.at[rl_v.at[j]], rgs[b], semsR[b])

        def wait_gather(j, b):
            if e_src is not None:
                pltpu.make_async_copy(e_src.at[sj_v.at[j]], egs[b],
                                      semsE[b]).wait()
            pltpu.make_async_copy(r_src.at[rl_v.at[j]], rgs[b],
                                  semsR[b]).wait()

        for phase in range(CHUNKS // PCH):
            stage_idx(phase)
            for b in range(NBUF - 1):
                issue(b, b)

            @pl.loop(0, PCH // NBUF)
            def _(g):
                for b in range(NBUF):
                    j = NBUF * g + b
                    p = b % 2

                    wait_gather(j, b)

                    @pl.when(j >= 2)
                    def _():
                        wait_scatter(p, j - 2)

                    crunch(j, b, p)

                    @pl.when(j + NBUF - 1 < PCH)
                    def _():
                        issue(j + NBUF - 1, (b + NBUF - 1) % NBUF)

                    pltpu.async_copy(prods[p], acc.at[ob_v.at[j]], sems_s[p],
                                     add=True)

            wait_scatter(0, PCH - 2)
            wait_scatter(1, PCH - 1)

    hvv = hv[...]

    for phase in range(CHUNKS // PCH):
        stage_idx(phase)

        @pl.loop(0, PCH)
        def _(j):
            def grp(k, m):
                sv = sj_v[j, pl.ds(k * 16, 16)]
                w = plsc.load_gather(bmv, [lax.shift_right_logical(sv, 5)])
                return m | lax.shift_right_logical(w, sv & 31)

            m = lax.fori_loop(0, C // 16, grp, jnp.zeros((16,), jnp.int32))
            flag = jnp.max(m & 1)

            @pl.when(flag != 0)
            def _():
                pltpu.sync_copy(r0_hbm.at[cid].at[rl_v.at[j]], rg0)

                @plsc.parallel_loop(0, C, unroll=8)
                def _(i):
                    sv = plsc.load_gather(
                        sj_v, [jnp.full((16,), j, jnp.int32),
                               jnp.full((16,), i, jnp.int32)])
                    pr0[i, pl.ds(0, 16)] = jnp.where(
                        sv == hvv, rg0[i, pl.ds(0, 16)], 0.0)

                pltpu.sync_copy(pr0, acc.at[ob_v.at[j]], add=True)

    plsc.subcore_barrier()

    @pl.loop(0, 8)
    def _(k):
        base = sid * RPT + k * 392
        pltpu.sync_copy(acc.at[pl.ds(base, 392)], wb)

        @pl.loop(0, 392)
        def _(r):
            row = base + r
            v = wb[r, pl.ds(0, 16)]
            v = v - jnp.where(av[...] == row, gv[...], 0.0)
            wb[r, pl.ds(0, 16)] = jnp.minimum(v, 1.0)

        pltpu.sync_copy(wb, e1_hbm.at[cid, pl.ds(base, 392)])
        pltpu.sync_copy(zb, acc.at[pl.ds(base, 392)])

    plsc.subcore_barrier()

    def crunch1(j, b, p):
        eg = egs[b]
        rg = rgs[b]
        pr = prods[p]

        @plsc.parallel_loop(0, C, unroll=8)
        def _(i):
            pr[i, pl.ds(0, 16)] = (eg[i, pl.ds(0, 16)]
                                   * rg[i, pl.ds(0, 16)])

    run_step(e1_hbm.at[cid], r1_hbm.at[cid], crunch1)
    plsc.subcore_barrier()

    @pl.when(sid == 15)
    def _():
        pltpu.sync_copy(zb.at[pl.ds(0, EP - NE)],
                        acc.at[pl.ds(NE, EP - NE)])

    pltpu.sync_copy(acc.at[pl.ds(sid * RPT, RPT)],
                    out_hbm.at[pl.ds(sid * RPT, RPT), pl.ds(cid * HB, HB)])


def _follow2(r2_0, r2_1, sj3, rl3, ob3, head_hb, ans_hb, gt_hb, bm):
    f = functools.partial(
        pl.kernel,
        out_type=(jax.ShapeDtypeStruct((EP, B), jnp.float32),
                  jax.ShapeDtypeStruct((2, EP, HB), jnp.float32)),
        mesh=plsc.VectorSubcoreMesh(core_axis_name="c", subcore_axis_name="s"),
        scratch_types=(
            [pltpu.VMEM((PCH, C), jnp.int32)] * 3
            + [pltpu.VMEM((C, HB), jnp.float32)] * (2 * NBUF + 2)
            + [pltpu.VMEM((392, HB), jnp.float32)] * 2
            + [pltpu.VMEM((HB,), jnp.int32),
               pltpu.VMEM((HB,), jnp.int32),
               pltpu.VMEM((HB,), jnp.float32),
               pltpu.VMEM((BMW,), jnp.int32),
               pltpu.VMEM_SHARED((EP, HB), jnp.float32)]
            + [pltpu.SemaphoreType.DMA] * (2 * NBUF + 2)
        ),
        compiler_params=pltpu.CompilerParams(use_tc_tiling_on_sc=False,
                                             needs_layout_passes=False),
    )(_follow_body)
    return f(r2_0, r2_1, sj3, rl3, ob3, head_hb, ans_hb, gt_hb, bm)


def _passa_body(p_ref, emb_ref, p1_ref, s_ref, ls_ref):
    j = pl.program_id(0)

    @pl.when(j == 0)
    def _():
        p1_ref[...] = jnp.zeros((B, D), jnp.float32)
        s_ref[...] = jnp.zeros((1, B), jnp.float32)
        ls_ref[...] = jnp.zeros((1, 1), jnp.float32)

    e = jnp.minimum(p_ref[...], 1.0)
    ls_ref[...] += jnp.sum(e * e)[None, None]
    s_ref[...] += jnp.sum(e, axis=0, keepdims=True)
    dnums = (((0,), (0,)), ((), ()))

    @pl.when(j < EGRID - 1)
    def _():
        p1_ref[...] += lax.dot_general(e, emb_ref[...], dnums,
                                       preferred_element_type=jnp.float32)

    @pl.when(j == EGRID - 1)
    def _():
        rows2 = lax.broadcasted_iota(jnp.int32, (EBLK, D), 0) + j * EBLK
        emb = jnp.where(rows2 < NE, emb_ref[...], 0.0)
        p1_ref[...] += lax.dot_general(e, emb, dnums,
                                       preferred_element_type=jnp.float32)


def _pass_a(parts, emb_p):
    return pl.pallas_call(
        _passa_body,
        grid=(EGRID,),
        in_specs=[
            pl.BlockSpec((EBLK, B), lambda j: (j, 0)),
            pl.BlockSpec((EBLK, D), lambda j: (j, 0)),
        ],
        out_specs=[
            pl.BlockSpec((B, D), lambda j: (0, 0)),
            pl.BlockSpec((1, B), lambda j: (0, 0)),
            pl.BlockSpec((1, 1), lambda j: (0, 0)),
        ],
        out_shape=[
            jax.ShapeDtypeStruct((B, D), jnp.float32),
            jax.ShapeDtypeStruct((1, B), jnp.float32),
            jax.ShapeDtypeStruct((1, 1), jnp.float32),
        ],
        compiler_params=pltpu.CompilerParams(
            dimension_semantics=("arbitrary",)),
    )(parts, emb_p)


def _passb_body(p1_ref, emb_ref, bias_ref, m_out, s_out):
    j = pl.program_id(0)

    @pl.when(j == 0)
    def _():
        m_out[...] = jnp.full((B, 1), -1e30, jnp.float32)
        s_out[...] = jnp.zeros((B, 1), jnp.float32)

    pred = lax.dot_general(p1_ref[...], emb_ref[...], (((1,), (1,)), ((), ())),
                           preferred_element_type=jnp.float32)
    pred = pred + bias_ref[...]
    cols = lax.broadcasted_iota(jnp.int32, (B, EBLK), 1) + j * EBLK
    pred = jnp.where(cols < NE, pred, -1e30)
    bm = jnp.max(pred, axis=1, keepdims=True)
    mnew = jnp.maximum(m_out[...], bm)
    s_out[...] = (s_out[...] * jnp.exp(m_out[...] - mnew)
                  + jnp.sum(jnp.exp(pred - mnew), axis=1, keepdims=True))
    m_out[...] = mnew


def _pass_b(p1n, emb_p, bias2):
    return pl.pallas_call(
        _passb_body,
        grid=(EGRID,),
        in_specs=[
            pl.BlockSpec((B, D), lambda j: (0, 0)),
            pl.BlockSpec((EBLK, D), lambda j: (j, 0)),
            pl.BlockSpec((1, EBLK), lambda j: (0, j)),
        ],
        out_specs=[
            pl.BlockSpec((B, 1), lambda j: (0, 0)),
            pl.BlockSpec((B, 1), lambda j: (0, 0)),
        ],
        out_shape=[
            jax.ShapeDtypeStruct((B, 1), jnp.float32),
            jax.ShapeDtypeStruct((B, 1), jnp.float32),
        ],
        compiler_params=pltpu.CompilerParams(
            dimension_semantics=("arbitrary",)),
    )(p1n, emb_p, bias2)


def kernel(questions, e_s, answers, subj_idx, rel_idx, obj_idx,
           W_step0, b_step0, W_step1, b_step1, W_cq, b_cq,
           rel_emb, ent_emb, ent_bias):
    q = questions[:, 0]
    q_emb = rel_emb[q]
    rd0 = jax.nn.softmax(jnp.tanh(q_emb @ W_step0 + b_step0) @ rel_emb.T, axis=1)
    rd1 = jax.nn.softmax(jnp.tanh(q_emb @ W_step1 + b_step1) @ rel_emb.T, axis=1)
    gt_mask = jnp.take_along_axis(rd0, questions, axis=1)[:, 0]
    prev_rel = jnp.argmax(rd0, axis=1)
    curr_rel = jnp.argmax(rd1, axis=1)
    cond = ((jnp.abs(prev_rel - curr_rel) == 1)
            & (jnp.mod(jnp.minimum(prev_rel, curr_rel), 2) == 0))
    num_self = ((prev_rel == 0).astype(jnp.int32)
                + (curr_rel == 0).astype(jnp.int32))
    kill = jnp.maximum(cond.astype(jnp.float32),
                       (num_self == 1).astype(jnp.float32))
    head_ids = jnp.argmax(e_s, axis=1).astype(jnp.int32)
    ans_ids = jnp.argmax(answers, axis=1).astype(jnp.int32)

    r2_0 = jnp.stack([rd0[:HB].T, rd0[HB:].T])
    r2_1 = jnp.stack([rd1[:HB].T, rd1[HB:].T])
    bias2 = ent_bias.reshape(1, NE)

    def mk_bitmap(h):
        hs = jnp.sort(h)
        uniq = jnp.concatenate([jnp.ones((1,), bool), hs[1:] != hs[:-1]])
        vals = jnp.where(uniq, jnp.left_shift(jnp.int32(1), hs % 32), 0)
        hit = (hs // 32)[:, None] == jnp.arange(BMW, dtype=jnp.int32)[None, :]
        return jnp.sum(jnp.where(hit, vals[:, None], 0), axis=0,
                       dtype=jnp.int32)

    bm = jnp.stack([mk_bitmap(head_ids[:HB]), mk_bitmap(head_ids[HB:])])

    pad = TP - T
    zpad = jnp.zeros((pad,), jnp.int32)
    sj3 = jnp.concatenate([subj_idx, zpad]).reshape(16, CHUNKS, C)
    rl3 = jnp.concatenate([rel_idx, zpad]).reshape(16, CHUNKS, C)
    opad = NE + (jnp.arange(pad, dtype=jnp.int32) % (EP - NE))
    ob3 = jnp.concatenate([obj_idx, opad]).reshape(16, CHUNKS, C)

    parts1, _e1 = _follow2(r2_0, r2_1, sj3, rl3, ob3,
                           head_ids.reshape(2, HB), ans_ids.reshape(2, HB),
                           gt_mask.reshape(2, HB), bm)

    p1_raw, s_sum, ls2 = _pass_a(parts1, ent_emb)

    barange = jnp.arange(B)
    eh = jnp.minimum(parts1[head_ids, barange], 1.0)
    ea = jnp.minimum(parts1[ans_ids, barange], 1.0)
    same = head_ids == ans_ids
    ls_corr = jnp.where(
        same,
        10.0 * ((1.0 - kill) * eh - 1.0) ** 2 - eh * eh,
        (jnp.square(1.0 - kill) - 1.0) * eh * eh
        + 10.0 * (ea - 1.0) ** 2 - ea * ea)
    loss_score = (ls2[0, 0] + jnp.sum(ls_corr)) * (1.0 / (B * NE))

    s_fin = s_sum[0] - kill * eh
    p1 = p1_raw + (-kill * eh)[:, None] * ent_emb[head_ids]
    p1n = p1 * (1.0 / (s_fin + 1e-6))[:, None]

    m_run, s_run = _pass_b(p1n, ent_emb, bias2)
    pred_ans = jnp.sum(p1n * ent_emb[ans_ids], axis=1) + ent_bias[ans_ids]
    loss_prob = -jnp.mean(pred_ans - m_run[:, 0] - jnp.log(s_run[:, 0]))

    return (loss_score, loss_prob)
